# baseline (speedup 1.0000x reference)
import functools

import jax
import jax.numpy as jnp
from jax import lax
from jax.experimental import pallas as pl
from jax.experimental.pallas import tpu as pltpu
from jax.experimental.pallas import tpu_sc as plsc

N_NODES = 10000
N_EDGES = 320000
D = 128
D_EDGE = 16

NC = 2
NS = 16
NW = NC * NS

BS = 40
E_PAD = 327680
NB_IDX = E_PAD // BS
CH = 8
BPW0 = 448
BPW1 = 64
CORE1_BASE = NS * BPW0

N_PAD = 10112
ROWS_PER_TILE = N_PAD // NS
DUMP_ROW = N_NODES


_EP_BLOCK = 4096


def _eproj_body(ea_ref, we_ref, out_ref):
    out_ref[...] = jnp.dot(ea_ref[...], we_ref[...],
                           preferred_element_type=jnp.float32)


def _eproj(edge_attr_pad, W_e):
    grid = E_PAD // _EP_BLOCK
    return pl.pallas_call(
        _eproj_body,
        grid=(grid,),
        in_specs=[
            pl.BlockSpec((_EP_BLOCK, D_EDGE), lambda i: (i, 0)),
            pl.BlockSpec((D_EDGE, D), lambda i: (0, 0)),
        ],
        out_specs=pl.BlockSpec((_EP_BLOCK, D), lambda i: (i, 0)),
        out_shape=jax.ShapeDtypeStruct((E_PAD, D), jnp.float32),
    )(edge_attr_pad, W_e)


def _sc_body(x_hbm, ep_hbm, src_hbm, dst_hbm, out_hbm,
             siA, siB, diA, diB, xb0, xb1, eb0, eb1, sb0, sb1,
             agg_sh, gs0, gs1, es0, es1, ss0, ss1):
    c = lax.axis_index("c")
    s = lax.axis_index("s")
    wid = s * NC + c
    sidx = (siA, siB)
    didx = (diA, diB)
    xb = (xb0, xb1)
    eb = (eb0, eb1)
    sb = (sb0, sb1)
    gsem = (gs0, gs1)
    esem = (es0, es1)
    ssem = (ss0, ss1)

    def zrow(r, _):
        for cc in range(D // 16):
            xb0[r, pl.ds(cc * 16, 16)] = jnp.zeros((16,), jnp.float32)
        return 0
    lax.fori_loop(0, BS, zrow, 0)

    off0 = s * ROWS_PER_TILE
    for i in range(ROWS_PER_TILE // BS):
        pltpu.sync_copy(xb0, agg_sh.at[pl.ds(off0 + i * BS, BS)])
    rem = ROWS_PER_TILE % BS
    if rem:
        pltpu.sync_copy(xb0.at[pl.ds(0, rem)],
                        agg_sh.at[pl.ds(off0 + ROWS_PER_TILE - rem, rem)])

    plsc.subcore_barrier()

    nbatch = jnp.where(c == 0, BPW0, BPW1)
    nchunk = nbatch // CH
    b0 = jnp.where(c == 0, s * BPW0, CORE1_BASE + s * BPW1)

    def load_chunk(t, tp):
        pltpu.sync_copy(src_hbm.at[pl.ds(b0 + t * CH, CH)], sidx[tp])
        pltpu.sync_copy(dst_hbm.at[pl.ds(b0 + t * CH, CH)], didx[tp])

    def start_ge(b, row_ref, par):
        pltpu.async_copy(x_hbm.at[row_ref], xb[par], gsem[par])
        pltpu.async_copy(ep_hbm.at[pl.ds((b0 + b) * BS, BS)],
                         eb[par], esem[par])

    def wait_ge(par):
        pltpu.make_async_copy(x_hbm.at[siA.at[0]], xb[par],
                              gsem[par]).wait()
        pltpu.make_async_copy(ep_hbm.at[pl.ds(0, BS)], eb[par],
                              esem[par]).wait()

    def wait_scat(par):
        pltpu.make_async_copy(sb[par], agg_sh.at[diA.at[0]],
                              ssem[par]).wait()

    load_chunk(0, 0)
    start_ge(0, sidx[0].at[0], 0)
    start_ge(1, sidx[0].at[1], 1)

    def body(tt, _):
        for tp in (0, 1):
            t = 2 * tt + tp
            for j in range(CH):
                b = t * CH + j
                par = j % 2
                if j == 0:
                    @pl.when(t < nchunk - 1)
                    def _ld():
                        load_chunk(t + 1, 1 - tp)
                wait_ge(par)

                @pl.when(b >= 2)
                def _ws():
                    wait_scat(par)

                def rrow(r, _):
                    for cc in range(D // 16):
                        sl = pl.ds(cc * 16, 16)
                        sb[par][r, sl] = jnp.maximum(
                            xb[par][r, sl] + eb[par][r, sl], 0.0)
                    return 0
                lax.fori_loop(0, BS, rrow, 0)

                pltpu.async_copy(sb[par], agg_sh.at[didx[tp].at[j]],
                                 ssem[par], add=True)

                @pl.when(b + 2 < nbatch)
                def _pf():
                    if j < CH - 2:
                        start_ge(b + 2, sidx[tp].at[j + 2], par)
                    else:
                        start_ge(b + 2, sidx[1 - tp].at[j + 2 - CH], par)
        return 0
    lax.fori_loop(0, nchunk // 2, body, 0)
    wait_scat(0)
    wait_scat(1)

    plsc.subcore_barrier()

    pltpu.sync_copy(agg_sh.at[pl.ds(off0, ROWS_PER_TILE)],
                    out_hbm.at[c, pl.ds(off0, ROWS_PER_TILE)])


def _sc_aggregate(x, eproj, srcb, dstb):
    mesh = plsc.VectorSubcoreMesh(core_axis_name="c", subcore_axis_name="s",
                                  num_cores=NC, num_subcores=NS)
    f = pl.kernel(
        _sc_body,
        out_type=jax.ShapeDtypeStruct((NC, N_PAD, D), jnp.float32),
        mesh=mesh,
        scratch_types=[
            pltpu.VMEM((CH, BS), jnp.int32),
            pltpu.VMEM((CH, BS), jnp.int32),
            pltpu.VMEM((CH, BS), jnp.int32),
            pltpu.VMEM((CH, BS), jnp.int32),
            pltpu.VMEM((BS, D), jnp.float32),
            pltpu.VMEM((BS, D), jnp.float32),
            pltpu.VMEM((BS, D), jnp.float32),
            pltpu.VMEM((BS, D), jnp.float32),
            pltpu.VMEM((BS, D), jnp.float32),
            pltpu.VMEM((BS, D), jnp.float32),
            pltpu.VMEM_SHARED((N_PAD, D), jnp.float32),
            pltpu.SemaphoreType.DMA,
            pltpu.SemaphoreType.DMA,
            pltpu.SemaphoreType.DMA,
            pltpu.SemaphoreType.DMA,
            pltpu.SemaphoreType.DMA,
            pltpu.SemaphoreType.DMA,
        ],
    )
    return f(x, eproj, srcb, dstb)


_EPI_BLOCK = 400


def _ln(h, g, b):
    mu = jnp.mean(h, axis=-1, keepdims=True)
    var = jnp.mean((h - mu) ** 2, axis=-1, keepdims=True)
    return (h - mu) * lax.rsqrt(var + 1e-5) * g + b


def _epi_body(a_ref, x_ref, wc_ref, bc_ref, g1_ref, b1_ref,
              wf1_ref, bf1_ref, wf2_ref, bf2_ref, g2_ref, b2_ref, out_ref):
    agg = a_ref[0] + a_ref[1]
    h = jnp.dot(agg, wc_ref[...], preferred_element_type=jnp.float32)
    h = h + bc_ref[...]
    h = _ln(h, g1_ref[...], b1_ref[...])
    y = x_ref[...] + jnp.maximum(h, 0.0)
    f = jnp.dot(y, wf1_ref[...], preferred_element_type=jnp.float32)
    f = jnp.maximum(f + bf1_ref[...], 0.0)
    f = jnp.dot(f, wf2_ref[...], preferred_element_type=jnp.float32)
    f = f + bf2_ref[...]
    out_ref[...] = _ln(y + f, g2_ref[...], b2_ref[...])


def _epilogue(aggp, x, W_conv, b_conv, g1, b1, W_f1, bf1, W_f2, bf2, g2, b2):
    grid = N_NODES // _EPI_BLOCK
    row = lambda i: (i, 0)
    full = lambda i: (0, 0)
    vec = lambda shape: pl.BlockSpec(shape, full)
    return pl.pallas_call(
        _epi_body,
        grid=(grid,),
        in_specs=[
            pl.BlockSpec((NC, _EPI_BLOCK, D), lambda i: (0, i, 0)),
            pl.BlockSpec((_EPI_BLOCK, D), row),
            vec((D, D)), vec((1, D)), vec((1, D)), vec((1, D)),
            vec((D, 2 * D)), vec((1, 2 * D)),
            vec((2 * D, D)), vec((1, D)),
            vec((1, D)), vec((1, D)),
        ],
        out_specs=pl.BlockSpec((_EPI_BLOCK, D), row),
        out_shape=jax.ShapeDtypeStruct((N_NODES, D), jnp.float32),
    )(aggp, x, W_conv, b_conv.reshape(1, D), g1.reshape(1, D),
      b1.reshape(1, D), W_f1, bf1.reshape(1, 2 * D), W_f2,
      bf2.reshape(1, D), g2.reshape(1, D), b2.reshape(1, D))


def kernel(x, edge_index, edge_attr, W_e, W_conv, b_conv, g1, b1,
           W_f1, bf1, W_f2, bf2, g2, b2):
    pad = E_PAD - N_EDGES
    src = jnp.concatenate(
        [edge_index[0].astype(jnp.int32),
         jnp.zeros((pad,), jnp.int32)]).reshape(NB_IDX, BS)
    dst = jnp.concatenate(
        [edge_index[1].astype(jnp.int32),
         jnp.full((pad,), DUMP_ROW, jnp.int32)]).reshape(NB_IDX, BS)
    ea_pad = jnp.pad(edge_attr, ((0, pad), (0, 0)))
    eproj = _eproj(ea_pad, W_e)
    aggp = _sc_aggregate(x, eproj, src, dst)
    return _epilogue(aggp, x, W_conv, b_conv, g1, b1,
                     W_f1, bf1, W_f2, bf2, g2, b2)


# speedup vs baseline: 1.0292x; 1.0292x over previous
import functools

import jax
import jax.numpy as jnp
from jax import lax
from jax.experimental import pallas as pl
from jax.experimental.pallas import tpu as pltpu
from jax.experimental.pallas import tpu_sc as plsc

N_NODES = 10000
N_EDGES = 320000
D = 128
D_EDGE = 16

NC = 2
NS = 16
NW = NC * NS

BS = 40
E_PAD = 327680
NB_IDX = E_PAD // BS
CH = 8
BPW0 = 432
BPW1 = 80
CORE1_BASE = NS * BPW0

N_PAD = 10112
ROWS_PER_TILE = N_PAD // NS
DUMP_ROW = N_NODES


_EP_BLOCK = 4096


def _eproj_body(ea_ref, we_ref, out_ref):
    out_ref[...] = jnp.dot(ea_ref[...], we_ref[...],
                           preferred_element_type=jnp.float32)


def _eproj(edge_attr_pad, W_e):
    grid = E_PAD // _EP_BLOCK
    return pl.pallas_call(
        _eproj_body,
        grid=(grid,),
        in_specs=[
            pl.BlockSpec((_EP_BLOCK, D_EDGE), lambda i: (i, 0)),
            pl.BlockSpec((D_EDGE, D), lambda i: (0, 0)),
        ],
        out_specs=pl.BlockSpec((_EP_BLOCK, D), lambda i: (i, 0)),
        out_shape=jax.ShapeDtypeStruct((E_PAD, D), jnp.float32),
    )(edge_attr_pad, W_e)


def _sc_body(x_hbm, ep_hbm, src_hbm, dst_hbm, out_hbm,
             siA, siB, diA, diB, xb0, xb1, eb0, eb1, sb0, sb1,
             agg_sh, gs0, gs1, es0, es1, ss0, ss1):
    c = lax.axis_index("c")
    s = lax.axis_index("s")
    wid = s * NC + c
    sidx = (siA, siB)
    didx = (diA, diB)
    xb = (xb0, xb1)
    eb = (eb0, eb1)
    sb = (sb0, sb1)
    gsem = (gs0, gs1)
    esem = (es0, es1)
    ssem = (ss0, ss1)

    def zrow(r, _):
        for cc in range(D // 16):
            xb0[r, pl.ds(cc * 16, 16)] = jnp.zeros((16,), jnp.float32)
        return 0
    lax.fori_loop(0, BS, zrow, 0)

    off0 = s * ROWS_PER_TILE
    for i in range(ROWS_PER_TILE // BS):
        pltpu.sync_copy(xb0, agg_sh.at[pl.ds(off0 + i * BS, BS)])
    rem = ROWS_PER_TILE % BS
    if rem:
        pltpu.sync_copy(xb0.at[pl.ds(0, rem)],
                        agg_sh.at[pl.ds(off0 + ROWS_PER_TILE - rem, rem)])

    plsc.subcore_barrier()

    nbatch = jnp.where(c == 0, BPW0, BPW1)
    nchunk = nbatch // CH
    b0 = jnp.where(c == 0, s * BPW0, CORE1_BASE + s * BPW1)

    def load_chunk(t, tp):
        pltpu.sync_copy(src_hbm.at[pl.ds(b0 + t * CH, CH)], sidx[tp])
        pltpu.sync_copy(dst_hbm.at[pl.ds(b0 + t * CH, CH)], didx[tp])

    def start_ge(b, row_ref, par):
        pltpu.async_copy(x_hbm.at[row_ref], xb[par], gsem[par])
        pltpu.async_copy(ep_hbm.at[pl.ds((b0 + b) * BS, BS)],
                         eb[par], esem[par])

    def wait_ge(par):
        pltpu.make_async_copy(x_hbm.at[siA.at[0]], xb[par],
                              gsem[par]).wait()
        pltpu.make_async_copy(ep_hbm.at[pl.ds(0, BS)], eb[par],
                              esem[par]).wait()

    def wait_scat(par):
        pltpu.make_async_copy(sb[par], agg_sh.at[diA.at[0]],
                              ssem[par]).wait()

    load_chunk(0, 0)
    start_ge(0, sidx[0].at[0], 0)
    start_ge(1, sidx[0].at[1], 1)

    def body(tt, _):
        for tp in (0, 1):
            t = 2 * tt + tp
            for j in range(CH):
                b = t * CH + j
                par = j % 2
                if j == 0:
                    @pl.when(t < nchunk - 1)
                    def _ld():
                        load_chunk(t + 1, 1 - tp)
                wait_ge(par)

                @pl.when(b >= 2)
                def _ws():
                    wait_scat(par)

                def rrow(r, _):
                    for cc in range(D // 16):
                        sl = pl.ds(cc * 16, 16)
                        sb[par][r, sl] = jnp.maximum(
                            xb[par][r, sl] + eb[par][r, sl], 0.0)
                    return 0
                lax.fori_loop(0, BS, rrow, 0)

                pltpu.async_copy(sb[par], agg_sh.at[didx[tp].at[j]],
                                 ssem[par], add=True)

                @pl.when(b + 2 < nbatch)
                def _pf():
                    if j < CH - 2:
                        start_ge(b + 2, sidx[tp].at[j + 2], par)
                    else:
                        start_ge(b + 2, sidx[1 - tp].at[j + 2 - CH], par)
        return 0
    lax.fori_loop(0, nchunk // 2, body, 0)
    wait_scat(0)
    wait_scat(1)

    plsc.subcore_barrier()

    pltpu.sync_copy(agg_sh.at[pl.ds(off0, ROWS_PER_TILE)],
                    out_hbm.at[c, pl.ds(off0, ROWS_PER_TILE)])


def _sc_aggregate(x, eproj, srcb, dstb):
    mesh = plsc.VectorSubcoreMesh(core_axis_name="c", subcore_axis_name="s",
                                  num_cores=NC, num_subcores=NS)
    f = pl.kernel(
        _sc_body,
        out_type=jax.ShapeDtypeStruct((NC, N_PAD, D), jnp.float32),
        mesh=mesh,
        scratch_types=[
            pltpu.VMEM((CH, BS), jnp.int32),
            pltpu.VMEM((CH, BS), jnp.int32),
            pltpu.VMEM((CH, BS), jnp.int32),
            pltpu.VMEM((CH, BS), jnp.int32),
            pltpu.VMEM((BS, D), jnp.float32),
            pltpu.VMEM((BS, D), jnp.float32),
            pltpu.VMEM((BS, D), jnp.float32),
            pltpu.VMEM((BS, D), jnp.float32),
            pltpu.VMEM((BS, D), jnp.float32),
            pltpu.VMEM((BS, D), jnp.float32),
            pltpu.VMEM_SHARED((N_PAD, D), jnp.float32),
            pltpu.SemaphoreType.DMA,
            pltpu.SemaphoreType.DMA,
            pltpu.SemaphoreType.DMA,
            pltpu.SemaphoreType.DMA,
            pltpu.SemaphoreType.DMA,
            pltpu.SemaphoreType.DMA,
        ],
    )
    return f(x, eproj, srcb, dstb)


_EPI_BLOCK = 400


def _ln(h, g, b):
    mu = jnp.mean(h, axis=-1, keepdims=True)
    var = jnp.mean((h - mu) ** 2, axis=-1, keepdims=True)
    return (h - mu) * lax.rsqrt(var + 1e-5) * g + b


def _epi_body(a_ref, x_ref, wc_ref, bc_ref, g1_ref, b1_ref,
              wf1_ref, bf1_ref, wf2_ref, bf2_ref, g2_ref, b2_ref, out_ref):
    agg = a_ref[0] + a_ref[1]
    h = jnp.dot(agg, wc_ref[...], preferred_element_type=jnp.float32)
    h = h + bc_ref[...]
    h = _ln(h, g1_ref[...], b1_ref[...])
    y = x_ref[...] + jnp.maximum(h, 0.0)
    f = jnp.dot(y, wf1_ref[...], preferred_element_type=jnp.float32)
    f = jnp.maximum(f + bf1_ref[...], 0.0)
    f = jnp.dot(f, wf2_ref[...], preferred_element_type=jnp.float32)
    f = f + bf2_ref[...]
    out_ref[...] = _ln(y + f, g2_ref[...], b2_ref[...])


def _epilogue(aggp, x, W_conv, b_conv, g1, b1, W_f1, bf1, W_f2, bf2, g2, b2):
    grid = N_NODES // _EPI_BLOCK
    row = lambda i: (i, 0)
    full = lambda i: (0, 0)
    vec = lambda shape: pl.BlockSpec(shape, full)
    return pl.pallas_call(
        _epi_body,
        grid=(grid,),
        in_specs=[
            pl.BlockSpec((NC, _EPI_BLOCK, D), lambda i: (0, i, 0)),
            pl.BlockSpec((_EPI_BLOCK, D), row),
            vec((D, D)), vec((1, D)), vec((1, D)), vec((1, D)),
            vec((D, 2 * D)), vec((1, 2 * D)),
            vec((2 * D, D)), vec((1, D)),
            vec((1, D)), vec((1, D)),
        ],
        out_specs=pl.BlockSpec((_EPI_BLOCK, D), row),
        out_shape=jax.ShapeDtypeStruct((N_NODES, D), jnp.float32),
    )(aggp, x, W_conv, b_conv.reshape(1, D), g1.reshape(1, D),
      b1.reshape(1, D), W_f1, bf1.reshape(1, 2 * D), W_f2,
      bf2.reshape(1, D), g2.reshape(1, D), b2.reshape(1, D))


def kernel(x, edge_index, edge_attr, W_e, W_conv, b_conv, g1, b1,
           W_f1, bf1, W_f2, bf2, g2, b2):
    pad = E_PAD - N_EDGES
    src = jnp.concatenate(
        [edge_index[0].astype(jnp.int32),
         jnp.zeros((pad,), jnp.int32)]).reshape(NB_IDX, BS)
    dst = jnp.concatenate(
        [edge_index[1].astype(jnp.int32),
         jnp.full((pad,), DUMP_ROW, jnp.int32)]).reshape(NB_IDX, BS)
    ea_pad = jnp.pad(edge_attr, ((0, pad), (0, 0)))
    eproj = _eproj(ea_pad, W_e)
    aggp = _sc_aggregate(x, eproj, src, dst)
    return _epilogue(aggp, x, W_conv, b_conv, g1, b1,
                     W_f1, bf1, W_f2, bf2, g2, b2)
